# baseline (speedup 1.0000x reference)
import functools

import jax
import jax.numpy as jnp
from jax import lax
from jax.experimental import pallas as pl
from jax.experimental.pallas import tpu as pltpu
from jax.experimental.pallas import tpu_sc as plsc

_N = 1600000
_TARGET_BER = 0.5
_NUM_CORES = 2
_NUM_SUBCORES = 16
_NUM_TILES = _NUM_CORES * _NUM_SUBCORES
_PER_TILE = _N // _NUM_TILES
_CHUNK = 10000
_STEPS = _CHUNK // 16

_mesh = plsc.VectorSubcoreMesh(
    core_axis_name="c", subcore_axis_name="s",
    num_cores=_NUM_CORES, num_subcores=_NUM_SUBCORES)


@functools.partial(
    pl.kernel,
    out_type=(
        jax.ShapeDtypeStruct((_NUM_TILES, 16), jnp.float32),
        jax.ShapeDtypeStruct((_NUM_TILES, 16), jnp.float32),
    ),
    mesh=_mesh,
    compiler_params=pltpu.CompilerParams(needs_layout_passes=False),
    scratch_types=[
        pltpu.VMEM((2 * _CHUNK,), jnp.float32),
        pltpu.VMEM((_CHUNK,), jnp.int32),
        pltpu.VMEM((_CHUNK,), jnp.int32),
        pltpu.VMEM((16,), jnp.float32),
        pltpu.VMEM((16,), jnp.float32),
    ],
)
def _partials_sc(in_hbm, tgt_hbm, sens_hbm, sums_hbm, cnts_hbm,
                 in_v, tgt_v, sens_v, res_s_v, res_c_v):
    wid = lax.axis_index("s") * _NUM_CORES + lax.axis_index("c")
    base = wid * _PER_TILE
    iota = lax.iota(jnp.int32, 16)
    zero = jnp.zeros((16,), jnp.float32)
    one = jnp.ones((16,), jnp.float32)

    accs = (zero, zero, zero, zero, zero, zero, zero, zero)
    for chunk in range(_PER_TILE // _CHUNK):
        start = base + chunk * _CHUNK
        pltpu.sync_copy(in_hbm.at[pl.ds(2 * start, 2 * _CHUNK)], in_v)
        pltpu.sync_copy(tgt_hbm.at[pl.ds(start, _CHUNK)], tgt_v)
        pltpu.sync_copy(sens_hbm.at[pl.ds(start, _CHUNK)], sens_v)

        @plsc.parallel_loop(0, _STEPS, unroll=8, carry=accs)
        def accs(i, carry):
            s0, s1, s2, s3, c0, c1, c2, c3 = carry
            off = i * 16
            t = tgt_v[pl.ds(off, 16)]
            s = sens_v[pl.ds(off, 16)]
            x = plsc.load_gather(in_v, [2 * (off + iota) + t])
            x = jnp.abs(jnp.float32(1.0) - x)
            m0 = s == 0
            m1 = s == 1
            m2 = s == 2
            m3 = s == 3
            s0 = s0 + jnp.where(m0, x, zero)
            s1 = s1 + jnp.where(m1, x, zero)
            s2 = s2 + jnp.where(m2, x, zero)
            s3 = s3 + jnp.where(m3, x, zero)
            c0 = c0 + jnp.where(m0, one, zero)
            c1 = c1 + jnp.where(m1, one, zero)
            c2 = c2 + jnp.where(m2, one, zero)
            c3 = c3 + jnp.where(m3, one, zero)
            return (s0, s1, s2, s3, c0, c1, c2, c3)

    res_s = zero
    res_c = zero
    for g in range(4):
        res_s = jnp.where(iota == g, jnp.sum(accs[g]), res_s)
        res_c = jnp.where(iota == g, jnp.sum(accs[4 + g]), res_c)
    res_s_v[...] = res_s
    res_c_v[...] = res_c
    pltpu.sync_copy(res_s_v, sums_hbm.at[wid])
    pltpu.sync_copy(res_c_v, cnts_hbm.at[wid])


def _finalize_tc(s_ref, c_ref, o_ref):
    ts = jnp.sum(s_ref[...], axis=0, keepdims=True)
    tc = jnp.sum(c_ref[...], axis=0, keepdims=True)
    present = tc > 0
    means = jnp.where(present, ts / jnp.maximum(tc, jnp.float32(1e-12)), 0.0)
    li = lax.broadcasted_iota(jnp.int32, (1, 16), 1)
    ng = jnp.max(jnp.where(present, li + 1, 0)).astype(jnp.float32)
    gm = jnp.sum(means) / ng
    o_ref[...] = jnp.reshape(jnp.abs(jnp.float32(_TARGET_BER) - gm), (1, 1))


def kernel(input, target, sens):
    sums, cnts = _partials_sc(input.reshape(-1), target, sens)
    res = pl.pallas_call(
        _finalize_tc,
        out_shape=jax.ShapeDtypeStruct((1, 1), jnp.float32),
    )(sums, cnts)
    return res[0, 0]


# speedup vs baseline: 2.8936x; 2.8936x over previous
import functools

import jax
import jax.numpy as jnp
from jax import lax
from jax.experimental import pallas as pl
from jax.experimental.pallas import tpu as pltpu
from jax.experimental.pallas import tpu_sc as plsc

_N = 1600000
_TARGET_BER = 0.5
_NUM_CORES = 2
_NUM_SUBCORES = 16
_NUM_TILES = _NUM_CORES * _NUM_SUBCORES
_PER_TILE = _N // _NUM_TILES
_B = 400
_STEPS = _B // 16
_CHUNKS = _PER_TILE // _B

_mesh = plsc.VectorSubcoreMesh(
    core_axis_name="c", subcore_axis_name="s",
    num_cores=_NUM_CORES, num_subcores=_NUM_SUBCORES)


@functools.partial(
    pl.kernel,
    out_type=(
        jax.ShapeDtypeStruct((_NUM_TILES, 16), jnp.float32),
        jax.ShapeDtypeStruct((_NUM_TILES, 16), jnp.float32),
    ),
    mesh=_mesh,
    compiler_params=pltpu.CompilerParams(needs_layout_passes=False),
    scratch_types=[
        pltpu.VMEM((_B, 2), jnp.float32),
        pltpu.VMEM((_B,), jnp.int32),
        pltpu.VMEM((_B,), jnp.int32),
        pltpu.VMEM((16,), jnp.float32),
        pltpu.VMEM((16,), jnp.float32),
        pltpu.SemaphoreType.DMA,
    ],
)
def _partials_sc(in_hbm, tgt_hbm, sens_hbm, sums_hbm, cnts_hbm,
                 rows_v, tgt_v, sens_v, res_s_v, res_c_v, sem):
    wid = lax.axis_index("s") * _NUM_CORES + lax.axis_index("c")
    base = wid * _PER_TILE
    iota = lax.iota(jnp.int32, 16)
    zero = jnp.zeros((16,), jnp.float32)
    one = jnp.ones((16,), jnp.float32)

    def chunk_body(chunk, accs):
        start = base + chunk * _B
        cp_in = pltpu.async_copy(in_hbm.at[pl.ds(start, _B)], rows_v, sem)
        cp_t = pltpu.async_copy(tgt_hbm.at[pl.ds(start, _B)], tgt_v, sem)
        cp_s = pltpu.async_copy(sens_hbm.at[pl.ds(start, _B)], sens_v, sem)
        cp_in.wait()
        cp_t.wait()
        cp_s.wait()

        @plsc.parallel_loop(0, _STEPS, unroll=5, carry=accs)
        def accs(j, carry):
            s0, s1, s2, s3, c0, c1, c2, c3 = carry
            off = j * 16
            t = tgt_v[pl.ds(off, 16)]
            s = sens_v[pl.ds(off, 16)]
            x = plsc.load_gather(rows_v, [off + iota, t])
            x = jnp.abs(jnp.float32(1.0) - x)
            m0 = s == 0
            m1 = s == 1
            m2 = s == 2
            m3 = s == 3
            s0 = s0 + jnp.where(m0, x, zero)
            s1 = s1 + jnp.where(m1, x, zero)
            s2 = s2 + jnp.where(m2, x, zero)
            s3 = s3 + jnp.where(m3, x, zero)
            c0 = c0 + jnp.where(m0, one, zero)
            c1 = c1 + jnp.where(m1, one, zero)
            c2 = c2 + jnp.where(m2, one, zero)
            c3 = c3 + jnp.where(m3, one, zero)
            return (s0, s1, s2, s3, c0, c1, c2, c3)

        return accs

    accs = lax.fori_loop(
        0, _CHUNKS, chunk_body,
        (zero, zero, zero, zero, zero, zero, zero, zero))

    res_s = zero
    res_c = zero
    for g in range(4):
        res_s = jnp.where(iota == g, jnp.sum(accs[g]), res_s)
        res_c = jnp.where(iota == g, jnp.sum(accs[4 + g]), res_c)
    res_s_v[...] = res_s
    res_c_v[...] = res_c
    pltpu.sync_copy(res_s_v, sums_hbm.at[wid])
    pltpu.sync_copy(res_c_v, cnts_hbm.at[wid])


def _finalize_tc(s_ref, c_ref, o_ref):
    ts = jnp.sum(s_ref[...], axis=0, keepdims=True)
    tc = jnp.sum(c_ref[...], axis=0, keepdims=True)
    present = tc > 0
    means = jnp.where(present, ts / jnp.maximum(tc, jnp.float32(1e-12)), 0.0)
    li = lax.broadcasted_iota(jnp.int32, (1, 16), 1)
    ng = jnp.max(jnp.where(present, li + 1, 0)).astype(jnp.float32)
    gm = jnp.sum(means) / ng
    o_ref[...] = jnp.reshape(jnp.abs(jnp.float32(_TARGET_BER) - gm), (1, 1))


def kernel(input, target, sens):
    sums, cnts = _partials_sc(input, target, sens)
    res = pl.pallas_call(
        _finalize_tc,
        out_shape=jax.ShapeDtypeStruct((1, 1), jnp.float32),
    )(sums, cnts)
    return res[0, 0]
